# baseline (speedup 1.0000x reference)
import functools

import jax
import jax.numpy as jnp
from jax import lax
from jax.experimental import pallas as pl
from jax.experimental.pallas import tpu as pltpu
from jax.experimental.pallas import tpu_sc as plsc

B = 8
N = 14
D = 512
P = N * N
ROWS = B * P
NW = 32
LPW = D // NW
TAU = 0.5 / P
BETA = 4.0
COEF = TAU * BETA / N


def _mask_body(in_hbm, out_hbm, x_v, y_v, wi_v, wj_v):
    wid = lax.axis_index("s") * 2 + lax.axis_index("c")
    col = wid * LPW
    pltpu.sync_copy(in_hbm.at[:, pl.ds(col, LPW)], x_v)

    for b in range(B):
        base = b * P

        def amax_body(p, carry):
            m, idx = carry
            v = x_v[base + p, :]
            pred = v > m
            m2 = jnp.where(pred, v, m)
            idx2 = jnp.where(pred, jnp.full((LPW,), p, jnp.int32), idx)
            return m2, idx2

        m0 = x_v[base, :]
        idx0 = jnp.zeros((LPW,), jnp.int32)
        _, idx = lax.fori_loop(1, P, amax_body, (m0, idx0))

        n_v = jnp.full((LPW,), N, jnp.int32)
        i_max = lax.div(idx, n_v).astype(jnp.float32)
        j_max = lax.rem(idx, n_v).astype(jnp.float32)

        coef_v = jnp.full((LPW,), COEF, jnp.float32)
        for k in range(N):
            kf = jnp.full((LPW,), float(k), jnp.float32)
            wi_v[k, :] = coef_v * jnp.abs(kf - i_max)
            wj_v[k, :] = coef_v * jnp.abs(kf - j_max)

        tau_v = jnp.full((LPW,), TAU, jnp.float32)
        ntau_v = jnp.full((LPW,), -TAU, jnp.float32)

        def row_body(i, _):
            ui = tau_v - wi_v[i, :]

            def col_body(j, _):
                r = base + i * N + j
                mask = jnp.maximum(ui - wj_v[j, :], ntau_v)
                y_v[r, :] = x_v[r, :] * mask
                return ()

            lax.fori_loop(0, N, col_body, ())
            return ()

        lax.fori_loop(0, N, row_body, ())

    pltpu.sync_copy(y_v, out_hbm.at[:, pl.ds(col, LPW)])


@jax.jit
def _mask_layer(flat):
    return pl.kernel(
        _mask_body,
        out_type=jax.ShapeDtypeStruct((ROWS, D), jnp.float32),
        mesh=plsc.VectorSubcoreMesh(core_axis_name="c", subcore_axis_name="s"),
        compiler_params=pltpu.CompilerParams(use_tc_tiling_on_sc=False),
        scratch_types=[
            pltpu.VMEM((ROWS, LPW), jnp.float32),
            pltpu.VMEM((ROWS, LPW), jnp.float32),
            pltpu.VMEM((N, LPW), jnp.float32),
            pltpu.VMEM((N, LPW), jnp.float32),
        ],
    )(flat)


def kernel(inputs):
    flat = inputs.reshape(ROWS, D)
    out = _mask_layer(flat)
    return out.reshape(B, N, N, D)


# speedup vs baseline: 1.3941x; 1.3941x over previous
import functools

import jax
import jax.numpy as jnp
from jax import lax
from jax.experimental import pallas as pl
from jax.experimental.pallas import tpu as pltpu
from jax.experimental.pallas import tpu_sc as plsc

B = 8
N = 14
D = 512
P = N * N
SEG = 4
PS = P // SEG
ROWS = B * P
NW = 32
LPW = D // NW
TAU = 0.5 / P
BETA = 4.0
COEF = TAU * BETA / N


def _mask_body(in_hbm, out_hbm, x_v, y_v, ui_v):
    wid = lax.axis_index("s") * 2 + lax.axis_index("c")
    col = wid * LPW
    pltpu.sync_copy(in_hbm.at[:, pl.ds(col, LPW)], x_v)

    neg_inf = jnp.full((LPW,), -jnp.inf, jnp.float32)
    zero_i = jnp.zeros((LPW,), jnp.int32)
    tau_v = jnp.full((LPW,), TAU, jnp.float32)
    ntau_v = jnp.full((LPW,), -TAU, jnp.float32)
    coef_v = jnp.full((LPW,), COEF, jnp.float32)

    for b in range(B):
        base = b * P

        def amax_body(t, carry):
            out = []
            tv = jnp.full((LPW,), t, jnp.int32)
            for s in range(SEG):
                m, idx = carry[2 * s], carry[2 * s + 1]
                v = x_v[base + s * PS + t, :]
                pred = v > m
                out.append(jnp.where(pred, v, m))
                out.append(jnp.where(pred, tv, idx))
            return tuple(out)

        init = (neg_inf, zero_i) * SEG
        fin = lax.fori_loop(0, PS, amax_body, init)
        m, idx = fin[0], fin[1]
        for s in range(1, SEG):
            ms = fin[2 * s]
            idxs = fin[2 * s + 1] + jnp.full((LPW,), s * PS, jnp.int32)
            pred = ms > m
            m = jnp.where(pred, ms, m)
            idx = jnp.where(pred, idxs, idx)

        n_v = jnp.full((LPW,), N, jnp.int32)
        i_max = lax.div(idx, n_v).astype(jnp.float32)
        j_max = lax.rem(idx, n_v).astype(jnp.float32)

        wj = []
        for k in range(N):
            kf = jnp.full((LPW,), float(k), jnp.float32)
            ui_v[k, :] = tau_v - coef_v * jnp.abs(kf - i_max)
            wj.append(coef_v * jnp.abs(kf - j_max))

        def row_body(i, _):
            r0 = base + i * N
            ui = ui_v[i, :]
            for j in range(N):
                mask = jnp.maximum(ui - wj[j], ntau_v)
                y_v[r0 + j, :] = x_v[r0 + j, :] * mask
            return ()

        lax.fori_loop(0, N, row_body, ())

    pltpu.sync_copy(y_v, out_hbm.at[:, pl.ds(col, LPW)])


@jax.jit
def _mask_layer(flat):
    return pl.kernel(
        _mask_body,
        out_type=jax.ShapeDtypeStruct((ROWS, D), jnp.float32),
        mesh=plsc.VectorSubcoreMesh(core_axis_name="c", subcore_axis_name="s"),
        compiler_params=pltpu.CompilerParams(use_tc_tiling_on_sc=False),
        scratch_types=[
            pltpu.VMEM((ROWS, LPW), jnp.float32),
            pltpu.VMEM((ROWS, LPW), jnp.float32),
            pltpu.VMEM((N, LPW), jnp.float32),
        ],
    )(flat)


def kernel(inputs):
    flat = inputs.reshape(ROWS, D)
    out = _mask_layer(flat)
    return out.reshape(B, N, N, D)


# speedup vs baseline: 1.4712x; 1.0553x over previous
import functools

import jax
import jax.numpy as jnp
from jax import lax
from jax.experimental import pallas as pl
from jax.experimental.pallas import tpu as pltpu
from jax.experimental.pallas import tpu_sc as plsc

B = 8
N = 14
D = 512
P = N * N
CB = 128
NG = CB // 16
NCH = 7
TAU = 0.5 / P
BETA = 4.0
COEF = TAU * BETA / N


def _mask_body(in_hbm, out_hbm, x_v, y_v, ui_v):
    wid = lax.axis_index("s") * 2 + lax.axis_index("c")
    b = lax.rem(wid, 8)
    cblk = lax.div(wid, 8)
    col = cblk * CB
    pltpu.sync_copy(in_hbm.at[b, :, :, pl.ds(col, CB)], x_v)

    neg_inf = jnp.full((16,), -jnp.inf, jnp.float32)
    zero_i = jnp.zeros((16,), jnp.int32)
    tau_v = jnp.full((16,), TAU, jnp.float32)
    ntau_v = jnp.full((16,), -TAU, jnp.float32)
    coef_v = jnp.full((16,), COEF, jnp.float32)
    n_v = jnp.full((16,), N, jnp.int32)

    for g in range(NG):
        cs = g * 16

        def make_amax(di):
            def amax_body(j, carry):
                out = []
                tv = jnp.full((16,), j + di * N, jnp.int32)
                for s in range(NCH):
                    m, idx = carry[2 * s], carry[2 * s + 1]
                    v = x_v[2 * s + di, j, pl.ds(cs, 16)]
                    pred = v > m
                    out.append(jnp.where(pred, v, m))
                    out.append(jnp.where(pred, tv, idx))
                return tuple(out)
            return amax_body

        carry = (neg_inf, zero_i) * NCH
        carry = lax.fori_loop(0, N, make_amax(0), carry)
        carry = lax.fori_loop(0, N, make_amax(1), carry)
        m, idx = carry[0], carry[1]
        for s in range(1, NCH):
            ms = carry[2 * s]
            idxs = carry[2 * s + 1] + jnp.full((16,), s * 2 * N, jnp.int32)
            pred = ms > m
            m = jnp.where(pred, ms, m)
            idx = jnp.where(pred, idxs, idx)

        i_max = lax.div(idx, n_v).astype(jnp.float32)
        j_max = lax.rem(idx, n_v).astype(jnp.float32)

        wj = []
        for k in range(N):
            kf = jnp.full((16,), float(k), jnp.float32)
            ui_v[k, :] = tau_v - coef_v * jnp.abs(kf - i_max)
            wj.append(coef_v * jnp.abs(kf - j_max))

        def row_body(i, _):
            ui = ui_v[i, :]
            for j in range(N):
                mask = jnp.maximum(ui - wj[j], ntau_v)
                y_v[i, j, pl.ds(cs, 16)] = x_v[i, j, pl.ds(cs, 16)] * mask
            return ()

        lax.fori_loop(0, N, row_body, ())

    pltpu.sync_copy(y_v, out_hbm.at[b, :, :, pl.ds(col, CB)])


@jax.jit
def _mask_layer(inputs):
    return pl.kernel(
        _mask_body,
        out_type=jax.ShapeDtypeStruct((B, N, N, D), jnp.float32),
        mesh=plsc.VectorSubcoreMesh(core_axis_name="c", subcore_axis_name="s"),
        scratch_types=[
            pltpu.VMEM((N, N, CB), jnp.float32),
            pltpu.VMEM((N, N, CB), jnp.float32),
            pltpu.VMEM((N, 16), jnp.float32),
        ],
    )(inputs)


def kernel(inputs):
    return _mask_layer(inputs)


# speedup vs baseline: 10.4284x; 7.0886x over previous
import jax
import jax.numpy as jnp
from jax.experimental import pallas as pl

B = 8
N = 14
D = 512
CB = 128
TAU = 0.5 / (N * N)
BETA = 4.0
COEF = TAU * BETA / N


def _mask_body(x_ref, o_ref):
    m = x_ref[0, 0]
    mi = jnp.zeros((B, CB), jnp.int32)
    for i in range(N):
        for j in range(N):
            if i == 0 and j == 0:
                continue
            v = x_ref[i, j]
            pred = v > m
            m = jnp.where(pred, v, m)
            mi = jnp.where(pred, jnp.full((B, CB), i * N + j, jnp.int32), mi)

    i_max = (mi // N).astype(jnp.float32)
    j_max = (mi % N).astype(jnp.float32)

    ui = [TAU - COEF * jnp.abs(float(i) - i_max) for i in range(N)]
    wj = [COEF * jnp.abs(float(j) - j_max) for j in range(N)]

    for i in range(N):
        for j in range(N):
            mask = jnp.maximum(ui[i] - wj[j], -TAU)
            o_ref[i, j] = x_ref[i, j] * mask


@jax.jit
def _mask_layer(inputs):
    xt = inputs.transpose(1, 2, 0, 3)
    out = pl.pallas_call(
        _mask_body,
        grid=(D // CB,),
        in_specs=[pl.BlockSpec((N, N, B, CB), lambda k: (0, 0, 0, k))],
        out_specs=pl.BlockSpec((N, N, B, CB), lambda k: (0, 0, 0, k)),
        out_shape=jax.ShapeDtypeStruct((N, N, B, D), jnp.float32),
    )(xt)
    return out.transpose(2, 0, 1, 3)


def kernel(inputs):
    return _mask_layer(inputs)


# speedup vs baseline: 14.6128x; 1.4013x over previous
import jax
import jax.numpy as jnp
from jax.experimental import pallas as pl

B = 8
N = 14
D = 512
CB = 256
TAU = 0.5 / (N * N)
BETA = 4.0
COEF = TAU * BETA / N


def _mask_body(x_ref, o_ref):
    m = x_ref[0, 0]
    mi = jnp.zeros((B, CB), jnp.int32)
    for i in range(N):
        for j in range(N):
            if i == 0 and j == 0:
                continue
            v = x_ref[i, j]
            pred = v > m
            m = jnp.where(pred, v, m)
            mi = jnp.where(pred, jnp.full((B, CB), i * N + j, jnp.int32), mi)

    i_max = (mi // N).astype(jnp.float32)
    j_max = (mi % N).astype(jnp.float32)

    ui = [TAU - COEF * jnp.abs(float(i) - i_max) for i in range(N)]
    wj = [COEF * jnp.abs(float(j) - j_max) for j in range(N)]

    for i in range(N):
        for j in range(N):
            mask = jnp.maximum(ui[i] - wj[j], -TAU)
            o_ref[i, j] = x_ref[i, j] * mask


@jax.jit
def _mask_layer(inputs):
    xt = inputs.transpose(1, 2, 0, 3)
    out = pl.pallas_call(
        _mask_body,
        grid=(D // CB,),
        in_specs=[pl.BlockSpec((N, N, B, CB), lambda k: (0, 0, 0, k))],
        out_specs=pl.BlockSpec((N, N, B, CB), lambda k: (0, 0, 0, k)),
        out_shape=jax.ShapeDtypeStruct((N, N, B, D), jnp.float32),
    )(xt)
    return out.transpose(2, 0, 1, 3)


def kernel(inputs):
    return _mask_layer(inputs)
